# baseline (speedup 1.0000x reference)
import functools

import jax
import jax.numpy as jnp
from jax import lax
from jax.experimental import pallas as pl
from jax.experimental.pallas import tpu as pltpu
from jax.experimental.pallas import tpu_sc as plsc

_N = 2048
_HALF = _N // 2
_MARGIN = 0.1
_L = 16
_NW = 16
_ROWS_PER_W = _N // _NW


def _sc_body(o_hbm, y_hbm, out_hbm, o_v, y_v, ob_v, yb_v, part_v, red_v,
             shared_v):
    w = lax.axis_index("s")
    base = w * _ROWS_PER_W
    pltpu.sync_copy(o_hbm, o_v.at[pl.ds(0, _N)])
    pltpu.sync_copy(o_hbm, o_v.at[pl.ds(_N, _N)])
    pltpu.sync_copy(y_hbm, y_v.at[pl.ds(0, _N)])
    pltpu.sync_copy(y_hbm, y_v.at[pl.ds(_N, _N)])

    lane_scaled = lax.iota(jnp.int32, _L) * _L
    for g in range(_ROWS_PER_W // _L):
        vor = o_v[pl.ds(base + g * _L, _L)]
        vyr = y_v[pl.ds(base + g * _L, _L)]
        for c in range(_L):
            idx = lane_scaled + (g * _L * _L + c)
            plsc.store_scatter(ob_v, [idx], vor)
            plsc.store_scatter(yb_v, [idx], vyr)

    zero = jnp.zeros((_L,), jnp.float32)
    ones = jnp.ones((_L,), jnp.float32)
    iota = lax.iota(jnp.int32, _L)

    def unit(o_rb, y_rb, colbase, acc, extra_mask=None, sign=1.0):
        acc_s, acc_v = acc
        vo = o_v[pl.ds(colbase, _L)]
        vy = y_v[pl.ds(colbase, _L)]
        dy = y_rb - vy
        do = o_rb - vo
        ds = jnp.where(dy > 0.0, do, -do)
        p = jnp.maximum(_MARGIN - ds, 0.0)
        valid = dy != 0.0
        if extra_mask is not None:
            valid = valid & extra_mask
        return (acc_s + jnp.where(valid, sign * p, 0.0),
                acc_v + jnp.where(valid, sign * ones, 0.0))

    def row_body(j, carry):
        r = base + j
        o_rb = ob_v[pl.ds(j * _L, _L)]
        y_rb = yb_v[pl.ds(j * _L, _L)]
        s0 = (r // _L) * _L + _L

        carry = unit(o_rb, y_rb, s0 - _L, carry, extra_mask=iota > (r % _L))

        def col_body(i, c2):
            return unit(o_rb, y_rb, s0 + i, c2)

        carry = plsc.parallel_loop(0, _HALF, _L, unroll=8,
                                   carry=carry)(col_body)

        thr = jnp.where(r < _HALF, _HALF + 1, _HALF)
        q = thr + r - (s0 + _HALF - _L)
        carry = unit(o_rb, y_rb, s0 + _HALF - _L, carry,
                     extra_mask=iota >= q, sign=-1.0)
        return carry

    acc = plsc.parallel_loop(0, _ROWS_PER_W, 1, unroll=1,
                             carry=(zero, zero))(row_body)

    part_v[0, :] = acc[0]
    part_v[1, :] = acc[1]
    pltpu.sync_copy(part_v, shared_v.at[w])
    plsc.subcore_barrier()

    @pl.when(w == 0)
    def _finalize():
        pltpu.sync_copy(shared_v, red_v)
        tot_mx = red_v[0, 0, :]
        tot_v = red_v[0, 1, :]
        for i in range(1, _NW):
            tot_mx = tot_mx + red_v[i, 0, :]
            tot_v = tot_v + red_v[i, 1, :]
        s_s = jnp.sum(tot_mx)
        v_s = jnp.sum(tot_v)
        num = jnp.full((_L,), s_s, jnp.float32)
        den = jnp.full((_L,), jnp.maximum(v_s, 1.0), jnp.float32)
        part_v[0, :] = num / den
        pltpu.sync_copy(part_v.at[0], out_hbm)


_sc_loss = functools.partial(
    pl.kernel,
    out_type=jax.ShapeDtypeStruct((_L,), jnp.float32),
    mesh=plsc.VectorSubcoreMesh(core_axis_name="c", subcore_axis_name="s",
                                num_cores=1),
    compiler_params=pltpu.CompilerParams(needs_layout_passes=False),
    scratch_types=[
        pltpu.VMEM((2 * _N,), jnp.float32),
        pltpu.VMEM((2 * _N,), jnp.float32),
        pltpu.VMEM((_ROWS_PER_W * _L,), jnp.float32),
        pltpu.VMEM((_ROWS_PER_W * _L,), jnp.float32),
        pltpu.VMEM((2, _L), jnp.float32),
        pltpu.VMEM((_NW, 2, _L), jnp.float32),
        pltpu.VMEM_SHARED((_NW, 2, _L), jnp.float32),
    ],
)(_sc_body)


def kernel(outputs, y):
    res = _sc_loss(outputs.reshape(_N), y.reshape(_N))
    return res[0]


# speedup vs baseline: 1.2780x; 1.2780x over previous
import functools

import jax
import jax.numpy as jnp
from jax import lax
from jax.experimental import pallas as pl
from jax.experimental.pallas import tpu as pltpu
from jax.experimental.pallas import tpu_sc as plsc

_N = 2048
_MARGIN = 0.1
_L = 16

_SC_ROWS = 256
_NW = 16
_ROWS_PER_W = _SC_ROWS // _NW
_SC_BASE = _N - _SC_ROWS

_TC_ROWS = _N - _SC_ROWS
_BLK = 256


def _sc_body(o_hbm, y_hbm, out_hbm, o_v, y_v, ob_v, yb_v, part_v):
    w = lax.axis_index("s")
    base = _SC_BASE + w * _ROWS_PER_W
    pltpu.sync_copy(o_hbm, o_v)
    pltpu.sync_copy(y_hbm, y_v)

    lane_scaled = lax.iota(jnp.int32, _L) * _L
    for g in range(_ROWS_PER_W // _L):
        vor = o_v[pl.ds(base + g * _L, _L)]
        vyr = y_v[pl.ds(base + g * _L, _L)]
        for c in range(_L):
            idx = lane_scaled + (g * _L * _L + c)
            plsc.store_scatter(ob_v, [idx], vor)
            plsc.store_scatter(yb_v, [idx], vyr)

    zero = jnp.zeros((_L,), jnp.float32)
    ones = jnp.ones((_L,), jnp.float32)

    def row_body(r, carry):
        o_rb = ob_v[pl.ds(r, _L)]
        y_rb = yb_v[pl.ds(r, _L)]

        def col_body(i, c2):
            acc_s, acc_c = c2
            vo = o_v[pl.ds(i, _L)]
            vy = y_v[pl.ds(i, _L)]
            dy = y_rb - vy
            do = o_rb - vo
            ds = jnp.where(dy > 0.0, do, -do)
            p = jnp.maximum(_MARGIN - ds, 0.0)
            valid = dy != 0.0
            return (acc_s + jnp.where(valid, p, 0.0),
                    acc_c + jnp.where(valid, ones, 0.0))

        return plsc.parallel_loop(0, _N, _L, unroll=8, carry=carry)(col_body)

    acc = plsc.parallel_loop(0, _ROWS_PER_W * _L, _L, unroll=1,
                             carry=(zero, zero))(row_body)
    part_v[0, :] = acc[0]
    part_v[1, :] = acc[1]
    pltpu.sync_copy(part_v, out_hbm.at[w])


_sc_pairs = functools.partial(
    pl.kernel,
    out_type=jax.ShapeDtypeStruct((_NW, 2, _L), jnp.float32),
    mesh=plsc.VectorSubcoreMesh(core_axis_name="c", subcore_axis_name="s",
                                num_cores=1),
    compiler_params=pltpu.CompilerParams(needs_layout_passes=False),
    scratch_types=[
        pltpu.VMEM((_N,), jnp.float32),
        pltpu.VMEM((_N,), jnp.float32),
        pltpu.VMEM((_ROWS_PER_W * _L,), jnp.float32),
        pltpu.VMEM((_ROWS_PER_W * _L,), jnp.float32),
        pltpu.VMEM((2, _L), jnp.float32),
    ],
)(_sc_body)


def _tc_body(orow_ref, yrow_ref, ocol_ref, ycol_ref, parts_ref, out_ref,
             acc_ref):
    i = pl.program_id(0)

    @pl.when(i == 0)
    def _init():
        acc_ref[0] = 0.0
        acc_ref[1] = 0.0

    orow = orow_ref[...]
    yrow = yrow_ref[...]
    oc = ocol_ref[...]
    yc = ycol_ref[...]
    dy = yrow - yc
    do = orow - oc
    t = jnp.sign(dy)
    per = jnp.maximum(0.0, _MARGIN - t * do)
    valid = dy != 0.0
    acc_ref[0] += jnp.sum(jnp.where(valid, per, 0.0))
    acc_ref[1] += jnp.sum(valid.astype(jnp.float32))

    @pl.when(i == pl.num_programs(0) - 1)
    def _fin():
        p = parts_ref[...]
        s = acc_ref[0] + jnp.sum(p[:, 0, :])
        c = acc_ref[1] + jnp.sum(p[:, 1, :])
        out_ref[...] = jnp.full((1, 1), s / jnp.maximum(c, 1.0),
                                dtype=jnp.float32)


def kernel(outputs, y):
    o = outputs.reshape(_N)
    yv = y.reshape(_N)
    parts = _sc_pairs(o, yv)

    o2 = outputs.reshape(_N, 1)
    y2 = y.reshape(_N, 1)
    oc = outputs.reshape(1, _N)
    yc = y.reshape(1, _N)
    res = pl.pallas_call(
        _tc_body,
        grid=(_TC_ROWS // _BLK,),
        in_specs=[
            pl.BlockSpec((_BLK, 1), lambda i: (i, 0)),
            pl.BlockSpec((_BLK, 1), lambda i: (i, 0)),
            pl.BlockSpec((1, _N), lambda i: (0, 0)),
            pl.BlockSpec((1, _N), lambda i: (0, 0)),
            pl.BlockSpec((_NW, 2, _L), lambda i: (0, 0, 0)),
        ],
        out_specs=pl.BlockSpec((1, 1), lambda i: (0, 0)),
        out_shape=jax.ShapeDtypeStruct((1, 1), jnp.float32),
        scratch_shapes=[pltpu.SMEM((2,), jnp.float32)],
    )(o2, y2, oc, yc, parts)
    return res.reshape(())


# speedup vs baseline: 1.6496x; 1.2908x over previous
import functools

import jax
import jax.numpy as jnp
from jax import lax
from jax.experimental import pallas as pl
from jax.experimental.pallas import tpu as pltpu
from jax.experimental.pallas import tpu_sc as plsc

_N = 2048
_MARGIN = 0.1
_L = 16

_SC_ROWS = 256
_NW = 16
_ROWS_PER_W = _SC_ROWS // _NW
_SC_BASE = _N - _SC_ROWS

_TC_ROWS = _N - _SC_ROWS
_BLK = 512


def _sc_body(o_hbm, y_hbm, out_hbm, o_v, y_v, ob_v, yb_v, part_v):
    w = lax.axis_index("s")
    base = _SC_BASE + w * _ROWS_PER_W
    pltpu.sync_copy(o_hbm, o_v)
    pltpu.sync_copy(y_hbm, y_v)

    lane_scaled = lax.iota(jnp.int32, _L) * _L
    for g in range(_ROWS_PER_W // _L):
        vor = o_v[pl.ds(base + g * _L, _L)]
        vyr = y_v[pl.ds(base + g * _L, _L)]
        for c in range(_L):
            idx = lane_scaled + (g * _L * _L + c)
            plsc.store_scatter(ob_v, [idx], vor)
            plsc.store_scatter(yb_v, [idx], vyr)

    zero = jnp.zeros((_L,), jnp.float32)
    ones = jnp.ones((_L,), jnp.float32)

    def row_body(r, carry):
        o_rb = ob_v[pl.ds(r, _L)]
        y_rb = yb_v[pl.ds(r, _L)]

        def col_body(i, c2):
            acc_s, acc_c = c2
            vo = o_v[pl.ds(i, _L)]
            vy = y_v[pl.ds(i, _L)]
            dy = y_rb - vy
            do = o_rb - vo
            ds = jnp.where(dy > 0.0, do, -do)
            p = jnp.maximum(_MARGIN - ds, 0.0)
            valid = dy != 0.0
            return (acc_s + jnp.where(valid, p, 0.0),
                    acc_c + jnp.where(valid, ones, 0.0))

        return plsc.parallel_loop(0, _N, _L, unroll=8, carry=carry)(col_body)

    acc = plsc.parallel_loop(0, _ROWS_PER_W * _L, _L, unroll=1,
                             carry=(zero, zero))(row_body)
    part_v[0, :] = acc[0]
    part_v[1, :] = acc[1]
    pltpu.sync_copy(part_v, out_hbm.at[w])


_sc_pairs = functools.partial(
    pl.kernel,
    out_type=jax.ShapeDtypeStruct((_NW, 2, _L), jnp.float32),
    mesh=plsc.VectorSubcoreMesh(core_axis_name="c", subcore_axis_name="s",
                                num_cores=1),
    compiler_params=pltpu.CompilerParams(needs_layout_passes=False),
    scratch_types=[
        pltpu.VMEM((_N,), jnp.float32),
        pltpu.VMEM((_N,), jnp.float32),
        pltpu.VMEM((_ROWS_PER_W * _L,), jnp.float32),
        pltpu.VMEM((_ROWS_PER_W * _L,), jnp.float32),
        pltpu.VMEM((2, _L), jnp.float32),
    ],
)(_sc_body)


def _tc_body(orow_ref, yrow_ref, ocol_ref, ycol_ref, out_ref, acc_ref):
    i = pl.program_id(0)

    @pl.when(i == 0)
    def _init():
        acc_ref[0] = 0.0
        acc_ref[1] = 0.0

    orow = orow_ref[...]
    yrow = yrow_ref[...]
    oc = ocol_ref[...]
    yc = ycol_ref[...]
    dy = yrow - yc
    do = orow - oc
    t = jnp.sign(dy)
    per = jnp.maximum(0.0, _MARGIN - t * do)
    valid = dy != 0.0
    acc_ref[0] += jnp.sum(jnp.where(valid, per, 0.0))
    acc_ref[1] += jnp.sum(valid.astype(jnp.float32))

    @pl.when(i == pl.num_programs(0) - 1)
    def _fin():
        out_ref[...] = jnp.stack([acc_ref[0], acc_ref[1]]).reshape(1, 2)


def _fin_body(tc_ref, parts_ref, out_ref):
    p = parts_ref[...]
    t = tc_ref[...]
    s = jnp.sum(p[:, 0, :]) + t[0, 0]
    c = jnp.sum(p[:, 1, :]) + t[0, 1]
    out_ref[...] = jnp.full((1, 1), s / jnp.maximum(c, 1.0), dtype=jnp.float32)


def kernel(outputs, y):
    o = outputs.reshape(_N)
    yv = y.reshape(_N)
    parts = _sc_pairs(o, yv)

    o2 = outputs.reshape(_N, 1)
    y2 = y.reshape(_N, 1)
    oc = outputs.reshape(1, _N)
    yc = y.reshape(1, _N)
    tc_part = pl.pallas_call(
        _tc_body,
        grid=(_TC_ROWS // _BLK,),
        in_specs=[
            pl.BlockSpec((_BLK, 1), lambda i: (i, 0)),
            pl.BlockSpec((_BLK, 1), lambda i: (i, 0)),
            pl.BlockSpec((1, _N), lambda i: (0, 0)),
            pl.BlockSpec((1, _N), lambda i: (0, 0)),
        ],
        out_specs=pl.BlockSpec((1, 2), lambda i: (0, 0)),
        out_shape=jax.ShapeDtypeStruct((1, 2), jnp.float32),
        scratch_shapes=[pltpu.SMEM((2,), jnp.float32)],
    )(o2, y2, oc, yc)

    res = pl.pallas_call(
        _fin_body,
        out_shape=jax.ShapeDtypeStruct((1, 1), jnp.float32),
    )(tc_part, parts)
    return res.reshape(())
